# baseline (speedup 1.0000x reference)
import functools

import jax
import jax.numpy as jnp
from jax.experimental import pallas as pl
from jax.experimental.pallas import tpu as pltpu

_ALPHA = 0.25
_BETA = 1.0 / 9
_MD_THRES = 0.5
_NEG_THRES = _MD_THRES - 0.1
_BIG_I32 = 2**30


def _body(T, TA, A, B, N, C,
          cls_ref, reg_ref, anc_ref, ann_ref,
          out_cls_ref, out_reg_ref,
          cache_ref, gtmax_ref, gtidx_ref, acc_ref):
    b = pl.program_id(0)
    s = pl.program_id(1)
    tile = jax.lax.rem(s, T)

    @pl.when(s == 0)
    def _init_image():
        gtmax_ref[...] = jnp.full((N, 1), -1.0, jnp.float32)
        gtidx_ref[...] = jnp.zeros((N, 1), jnp.int32)
        acc_ref[0] = 0.0
        acc_ref[1] = 0.0
        acc_ref[2] = 0.0

    @pl.when((s == 0) & (b == 0))
    def _init_batch():
        acc_ref[3] = 0.0
        acc_ref[4] = 0.0

    ann = ann_ref[0]
    gcx = (ann[:, 0:1] + ann[:, 2:3]) * 0.5
    gcy = (ann[:, 1:2] + ann[:, 3:4]) * 0.5
    gw = ann[:, 2:3] - ann[:, 0:1]
    gh = ann[:, 3:4] - ann[:, 1:2]
    gang = ann[:, 4:5]
    glbl = ann[:, 5:6]

    laneid = jax.lax.broadcasted_iota(jnp.int32, (1, TA), 1)
    gid = laneid + tile * TA
    valid = gid < A

    @pl.when(s < T)
    def _pass1():
        gx1 = gcx - gw * 0.5
        gx2 = gcx + gw * 0.5
        gy1 = gcy - gh * 0.5
        gy2 = gcy + gh * 0.5
        area_g = (gx2 - gx1) * (gy2 - gy1)

        anc = jnp.transpose(anc_ref[0])
        acx = (anc[0:1, :] + anc[2:3, :]) * 0.5
        acy = (anc[1:2, :] + anc[3:4, :]) * 0.5
        aw = anc[2:3, :] - anc[0:1, :]
        ah = anc[3:4, :] - anc[1:2, :]
        aang = anc[4:5, :]
        ax1 = acx - aw * 0.5
        ax2 = acx + aw * 0.5
        ay1 = acy - ah * 0.5
        ay2 = acy + ah * 0.5
        area_a = (ax2 - ax1) * (ay2 - ay1)

        ix1 = jnp.maximum(ax1, gx1)
        iy1 = jnp.maximum(ay1, gy1)
        ix2 = jnp.minimum(ax2, gx2)
        iy2 = jnp.minimum(ay2, gy2)
        iw = jnp.clip(ix2 - ix1, 0.0)
        ih = jnp.clip(iy2 - iy1, 0.0)
        inter = iw * ih
        ua = area_a + area_g - inter
        iou = inter / jnp.maximum(ua, 1e-8)
        iou = jnp.where(valid, iou, 0.0)

        lane_nt = jax.lax.broadcasted_iota(jnp.int32, (N, TA), 1)
        colmax = jnp.max(iou, axis=1, keepdims=True)
        colarg = jnp.min(jnp.where(iou == colmax, lane_nt, _BIG_I32),
                         axis=1, keepdims=True) + tile * TA
        old = gtmax_ref[...]
        upd = colmax > old
        gtmax_ref[...] = jnp.where(upd, colmax, old)
        gtidx_ref[...] = jnp.where(upd, colarg, gtidx_ref[...])

        gt_rows = jax.lax.broadcasted_iota(jnp.int32, (N, TA), 0)
        iou_max = jnp.max(iou, axis=0, keepdims=True)
        iou_arg = jnp.min(jnp.where(iou == iou_max, gt_rows, _BIG_I32),
                          axis=0, keepdims=True)
        cache_ref[tile] = jnp.concatenate(
            [acx, acy, aw, ah, aang, iou_max,
             iou_arg.astype(jnp.float32), jnp.zeros((1, TA), jnp.float32)],
            axis=0)

    @pl.when(s >= T)
    def _pass2():
        slab = cache_ref[tile]
        acx, acy = slab[0:1, :], slab[1:2, :]
        aw, ah = slab[2:3, :], slab[3:4, :]
        aang = slab[4:5, :]
        iou_max = slab[5:6, :]
        iou_arg = slab[6:7, :].astype(jnp.int32)

        forced = jnp.any((gtidx_ref[...] == gid) & (gtmax_ref[...] < _MD_THRES),
                         axis=0, keepdims=True)
        positive = ((iou_max >= _MD_THRES) | forced) & valid

        gt_row = jax.lax.broadcasted_iota(jnp.int32, (N, 1), 0)
        onehot = (gt_row == iou_arg).astype(jnp.float32)
        tblT = jnp.transpose(jnp.concatenate(
            [gcx, gcy, gw, gh, gang, glbl], axis=1))
        assigned = jax.lax.dot_general(
            tblT, onehot, (((1,), (0,)), ((), ())),
            preferred_element_type=jnp.float32)
        a_cx, a_cy = assigned[0:1, :], assigned[1:2, :]
        a_w, a_h = assigned[2:3, :], assigned[3:4, :]
        a_ang = assigned[4:5, :]
        lbl = (assigned[5:6, :] + 0.5).astype(jnp.int32)

        c = jnp.clip(jnp.transpose(cls_ref[0]), 1e-4, 1.0 - 1e-4)
        f0 = ((1.0 - _ALPHA) * c * c) * (-jnp.log(1.0 - c + 1e-6))
        S0 = jnp.sum(f0, axis=0, keepdims=True)
        cid = jax.lax.broadcasted_iota(jnp.int32, (C, 1), 0)
        c_lbl = jnp.sum(jnp.where(lbl == cid, c, 0.0), axis=0, keepdims=True)
        f0_lbl = ((1.0 - _ALPHA) * c_lbl * c_lbl) * (-jnp.log(1.0 - c_lbl + 1e-6))
        om = 1.0 - c_lbl
        f1_lbl = (_ALPHA * om * om) * (-jnp.log(c_lbl + 1e-6))
        include = ((iou_max < _NEG_THRES) | positive) & valid
        cls_part = (jnp.sum(jnp.where(include, S0, 0.0))
                    + jnp.sum(jnp.where(positive, f1_lbl - f0_lbl, 0.0)))

        reg = jnp.transpose(reg_ref[0])
        dx = (a_cx - acx) / aw
        dy = (a_cy - acy) / ah
        dwc = jnp.log(a_w / aw)
        dhc = jnp.log(a_h / ah)
        dt = (a_ang - aang) * 3.141592653589793 / 180.0
        rt = jnp.concatenate([dx, dy, dwc, dhc, dt], axis=0)
        d = jnp.abs(reg - rt)
        rl = jnp.where(d < _BETA, 0.5 * d * d / _BETA, d - 0.5 * _BETA)
        rl = jnp.where(positive, rl, 0.0)

        acc_ref[0] = acc_ref[0] + cls_part
        acc_ref[1] = acc_ref[1] + jnp.sum(rl)
        acc_ref[2] = acc_ref[2] + jnp.sum(positive.astype(jnp.float32))

    @pl.when(s == 2 * T - 1)
    def _finish_image():
        npos = acc_ref[2]
        den = jnp.maximum(npos, 1.0)
        acc_ref[3] = acc_ref[3] + acc_ref[0] / den
        acc_ref[4] = acc_ref[4] + jnp.where(npos > 0.0,
                                            acc_ref[1] / (den * 5.0), 0.0)

    @pl.when((s == 2 * T - 1) & (b == B - 1))
    def _write_out():
        out_cls_ref[...] = jnp.full((1, 1), acc_ref[3] / B, jnp.float32)
        out_reg_ref[...] = jnp.full((1, 1), acc_ref[4] / B, jnp.float32)


def kernel(classifications, regressions, anchors, refined_achors, annotations):
    del refined_achors
    B, A, C = classifications.shape
    N = annotations.shape[1]
    TA = 16384 if A >= 16384 else A
    T = (A + TA - 1) // TA

    body = functools.partial(_body, T, TA, A, B, N, C)
    out_cls, out_reg = pl.pallas_call(
        body,
        grid=(B, 2 * T),
        in_specs=[
            pl.BlockSpec((1, TA, C), lambda b, s: (b, jnp.maximum(s - T, 0), 0)),
            pl.BlockSpec((1, TA, 5), lambda b, s: (b, jnp.maximum(s - T, 0), 0)),
            pl.BlockSpec((1, TA, 5), lambda b, s: (b, jnp.minimum(s, T - 1), 0)),
            pl.BlockSpec((1, N, 6), lambda b, s: (b, 0, 0)),
        ],
        out_specs=[
            pl.BlockSpec((1, 1), lambda b, s: (0, 0)),
            pl.BlockSpec((1, 1), lambda b, s: (0, 0)),
        ],
        out_shape=[
            jax.ShapeDtypeStruct((1, 1), jnp.float32),
            jax.ShapeDtypeStruct((1, 1), jnp.float32),
        ],
        scratch_shapes=[
            pltpu.VMEM((T, 8, TA), jnp.float32),
            pltpu.VMEM((N, 1), jnp.float32),
            pltpu.VMEM((N, 1), jnp.int32),
            pltpu.SMEM((8,), jnp.float32),
        ],
    )(classifications, regressions, anchors, annotations)
    return (out_cls.reshape(1), out_reg.reshape(1))


# speedup vs baseline: 1.1000x; 1.1000x over previous
import functools

import jax
import jax.numpy as jnp
from jax.experimental import pallas as pl
from jax.experimental.pallas import tpu as pltpu

_ALPHA = 0.25
_BETA = 1.0 / 9
_MD_THRES = 0.5
_NEG_THRES = _MD_THRES - 0.1
_BIG_I32 = 2**30


def _body(T, TA, A, B, N, C,
          cls_ref, reg_ref, anc_ref, ann_ref,
          out_cls_ref, out_reg_ref,
          cache_ref, gtmax_ref, gtidx_ref, acc_ref):
    b = pl.program_id(0)
    s = pl.program_id(1)
    tile = jax.lax.rem(s, T)

    @pl.when(s == 0)
    def _init_image():
        gtmax_ref[...] = jnp.full((N, 1), -1.0, jnp.float32)
        gtidx_ref[...] = jnp.zeros((N, 1), jnp.int32)
        acc_ref[0] = 0.0
        acc_ref[1] = 0.0
        acc_ref[2] = 0.0

    @pl.when((s == 0) & (b == 0))
    def _init_batch():
        acc_ref[3] = 0.0
        acc_ref[4] = 0.0

    ann = ann_ref[0]
    gcx = (ann[:, 0:1] + ann[:, 2:3]) * 0.5
    gcy = (ann[:, 1:2] + ann[:, 3:4]) * 0.5
    gw = ann[:, 2:3] - ann[:, 0:1]
    gh = ann[:, 3:4] - ann[:, 1:2]
    gang = ann[:, 4:5]
    glbl = ann[:, 5:6]

    laneid = jax.lax.broadcasted_iota(jnp.int32, (1, TA), 1)
    gid = laneid + tile * TA
    valid = gid < A

    @pl.when(s < T)
    def _pass1():
        gx1 = gcx - gw * 0.5
        gx2 = gcx + gw * 0.5
        gy1 = gcy - gh * 0.5
        gy2 = gcy + gh * 0.5
        area_g = (gx2 - gx1) * (gy2 - gy1)

        anc = jnp.transpose(anc_ref[0])
        acx = (anc[0:1, :] + anc[2:3, :]) * 0.5
        acy = (anc[1:2, :] + anc[3:4, :]) * 0.5
        aw = anc[2:3, :] - anc[0:1, :]
        ah = anc[3:4, :] - anc[1:2, :]
        aang = anc[4:5, :]
        ax1 = acx - aw * 0.5
        ax2 = acx + aw * 0.5
        ay1 = acy - ah * 0.5
        ay2 = acy + ah * 0.5
        area_a = (ax2 - ax1) * (ay2 - ay1)

        ix1 = jnp.maximum(ax1, gx1)
        iy1 = jnp.maximum(ay1, gy1)
        ix2 = jnp.minimum(ax2, gx2)
        iy2 = jnp.minimum(ay2, gy2)
        iw = jnp.clip(ix2 - ix1, 0.0)
        ih = jnp.clip(iy2 - iy1, 0.0)
        inter = iw * ih
        ua = area_a + area_g - inter
        iou = inter / jnp.maximum(ua, 1e-8)
        iou = jnp.where(valid, iou, 0.0)

        lane_nt = jax.lax.broadcasted_iota(jnp.int32, (N, TA), 1)
        colmax = jnp.max(iou, axis=1, keepdims=True)
        colarg = jnp.min(jnp.where(iou == colmax, lane_nt, _BIG_I32),
                         axis=1, keepdims=True) + tile * TA
        old = gtmax_ref[...]
        upd = colmax > old
        gtmax_ref[...] = jnp.where(upd, colmax, old)
        gtidx_ref[...] = jnp.where(upd, colarg, gtidx_ref[...])

        gt_rows = jax.lax.broadcasted_iota(jnp.int32, (N, TA), 0)
        iou_max = jnp.max(iou, axis=0, keepdims=True)
        iou_arg = jnp.min(jnp.where(iou == iou_max, gt_rows, _BIG_I32),
                          axis=0, keepdims=True)
        cache_ref[tile] = jnp.concatenate(
            [acx, acy, aw, ah, aang, iou_max,
             iou_arg.astype(jnp.float32), jnp.zeros((1, TA), jnp.float32)],
            axis=0)

    @pl.when(s >= T)
    def _pass2():
        slab = cache_ref[tile]
        acx, acy = slab[0:1, :], slab[1:2, :]
        aw, ah = slab[2:3, :], slab[3:4, :]
        aang = slab[4:5, :]
        iou_max = slab[5:6, :]
        iou_arg = slab[6:7, :].astype(jnp.int32)

        forced = jnp.any((gtidx_ref[...] == gid) & (gtmax_ref[...] < _MD_THRES),
                         axis=0, keepdims=True)
        positive = ((iou_max >= _MD_THRES) | forced) & valid

        gt_row = jax.lax.broadcasted_iota(jnp.int32, (N, 1), 0)
        onehot = (gt_row == iou_arg).astype(jnp.float32)
        tblT = jnp.transpose(jnp.concatenate(
            [gcx, gcy, gw, gh, gang, glbl], axis=1))
        assigned = jax.lax.dot_general(
            tblT, onehot, (((1,), (0,)), ((), ())),
            preferred_element_type=jnp.float32)
        a_cx, a_cy = assigned[0:1, :], assigned[1:2, :]
        a_w, a_h = assigned[2:3, :], assigned[3:4, :]
        a_ang = assigned[4:5, :]
        lbl = (assigned[5:6, :] + 0.5).astype(jnp.int32)

        c = jnp.clip(jnp.transpose(cls_ref[0]), 1e-4, 1.0 - 1e-4)
        f0 = ((1.0 - _ALPHA) * c * c) * (-jnp.log(1.0 - c + 1e-6))
        S0 = jnp.sum(f0, axis=0, keepdims=True)
        cid = jax.lax.broadcasted_iota(jnp.int32, (C, 1), 0)
        c_lbl = jnp.sum(jnp.where(lbl == cid, c, 0.0), axis=0, keepdims=True)
        f0_lbl = ((1.0 - _ALPHA) * c_lbl * c_lbl) * (-jnp.log(1.0 - c_lbl + 1e-6))
        om = 1.0 - c_lbl
        f1_lbl = (_ALPHA * om * om) * (-jnp.log(c_lbl + 1e-6))
        include = ((iou_max < _NEG_THRES) | positive) & valid
        cls_part = (jnp.sum(jnp.where(include, S0, 0.0))
                    + jnp.sum(jnp.where(positive, f1_lbl - f0_lbl, 0.0)))

        reg = jnp.transpose(reg_ref[0])
        dx = (a_cx - acx) / aw
        dy = (a_cy - acy) / ah
        dwc = jnp.log(a_w / aw)
        dhc = jnp.log(a_h / ah)
        dt = (a_ang - aang) * 3.141592653589793 / 180.0
        rt = jnp.concatenate([dx, dy, dwc, dhc, dt], axis=0)
        d = jnp.abs(reg - rt)
        rl = jnp.where(d < _BETA, 0.5 * d * d / _BETA, d - 0.5 * _BETA)
        rl = jnp.where(positive, rl, 0.0)

        acc_ref[0] = acc_ref[0] + cls_part
        acc_ref[1] = acc_ref[1] + jnp.sum(rl)
        acc_ref[2] = acc_ref[2] + jnp.sum(positive.astype(jnp.float32))

    @pl.when(s == 2 * T - 1)
    def _finish_image():
        npos = acc_ref[2]
        den = jnp.maximum(npos, 1.0)
        acc_ref[3] = acc_ref[3] + acc_ref[0] / den
        acc_ref[4] = acc_ref[4] + jnp.where(npos > 0.0,
                                            acc_ref[1] / (den * 5.0), 0.0)

    @pl.when((s == 2 * T - 1) & (b == B - 1))
    def _write_out():
        out_cls_ref[...] = jnp.full((1, 1), acc_ref[3] / B, jnp.float32)
        out_reg_ref[...] = jnp.full((1, 1), acc_ref[4] / B, jnp.float32)


def kernel(classifications, regressions, anchors, refined_achors, annotations):
    del refined_achors
    B, A, C = classifications.shape
    N = annotations.shape[1]
    TA = 8192 if A >= 8192 else A
    T = (A + TA - 1) // TA

    body = functools.partial(_body, T, TA, A, B, N, C)
    out_cls, out_reg = pl.pallas_call(
        body,
        grid=(B, 2 * T),
        in_specs=[
            pl.BlockSpec((1, TA, C), lambda b, s: (b, jnp.maximum(s - T, 0), 0)),
            pl.BlockSpec((1, TA, 5), lambda b, s: (b, jnp.maximum(s - T, 0), 0)),
            pl.BlockSpec((1, TA, 5), lambda b, s: (b, jnp.minimum(s, T - 1), 0)),
            pl.BlockSpec((1, N, 6), lambda b, s: (b, 0, 0)),
        ],
        out_specs=[
            pl.BlockSpec((1, 1), lambda b, s: (0, 0)),
            pl.BlockSpec((1, 1), lambda b, s: (0, 0)),
        ],
        out_shape=[
            jax.ShapeDtypeStruct((1, 1), jnp.float32),
            jax.ShapeDtypeStruct((1, 1), jnp.float32),
        ],
        scratch_shapes=[
            pltpu.VMEM((T, 8, TA), jnp.float32),
            pltpu.VMEM((N, 1), jnp.float32),
            pltpu.VMEM((N, 1), jnp.int32),
            pltpu.SMEM((8,), jnp.float32),
        ],
    )(classifications, regressions, anchors, annotations)
    return (out_cls.reshape(1), out_reg.reshape(1))


# speedup vs baseline: 1.1672x; 1.0611x over previous
import functools

import jax
import jax.numpy as jnp
from jax.experimental import pallas as pl
from jax.experimental.pallas import tpu as pltpu

_ALPHA = 0.25
_BETA = 1.0 / 9
_MD_THRES = 0.5
_NEG_THRES = _MD_THRES - 0.1
_BIG_I32 = 2**30


def _body(T, TA, A, B, N, C,
          cls_ref, reg_ref, anc_ref, ann_ref,
          out_cls_ref, out_reg_ref,
          cache_ref, gtmax_ref, gtidx_ref, acc_ref):
    b = pl.program_id(0)
    s = pl.program_id(1)
    tile = jax.lax.rem(s, T)

    @pl.when(s == 0)
    def _init_image():
        gtmax_ref[...] = jnp.full((N, 1), -1.0, jnp.float32)
        gtidx_ref[...] = jnp.zeros((N, 1), jnp.int32)
        acc_ref[0] = 0.0
        acc_ref[1] = 0.0
        acc_ref[2] = 0.0

    @pl.when((s == 0) & (b == 0))
    def _init_batch():
        acc_ref[3] = 0.0
        acc_ref[4] = 0.0

    ann = ann_ref[0]
    gcx = (ann[:, 0:1] + ann[:, 2:3]) * 0.5
    gcy = (ann[:, 1:2] + ann[:, 3:4]) * 0.5
    gw = ann[:, 2:3] - ann[:, 0:1]
    gh = ann[:, 3:4] - ann[:, 1:2]
    gang = ann[:, 4:5]
    glbl = ann[:, 5:6]

    laneid = jax.lax.broadcasted_iota(jnp.int32, (1, TA), 1)
    gid = laneid + tile * TA
    valid = gid < A

    @pl.when(s < T)
    def _pass1():
        gx1 = gcx - gw * 0.5
        gx2 = gcx + gw * 0.5
        gy1 = gcy - gh * 0.5
        gy2 = gcy + gh * 0.5
        area_g = (gx2 - gx1) * (gy2 - gy1)

        anc = jnp.transpose(anc_ref[0])
        acx = (anc[0:1, :] + anc[2:3, :]) * 0.5
        acy = (anc[1:2, :] + anc[3:4, :]) * 0.5
        aw = anc[2:3, :] - anc[0:1, :]
        ah = anc[3:4, :] - anc[1:2, :]
        aang = anc[4:5, :]
        ax1 = acx - aw * 0.5
        ax2 = acx + aw * 0.5
        ay1 = acy - ah * 0.5
        ay2 = acy + ah * 0.5
        area_a = (ax2 - ax1) * (ay2 - ay1)

        ix1 = jnp.maximum(ax1, gx1)
        iy1 = jnp.maximum(ay1, gy1)
        ix2 = jnp.minimum(ax2, gx2)
        iy2 = jnp.minimum(ay2, gy2)
        iw = jnp.clip(ix2 - ix1, 0.0)
        ih = jnp.clip(iy2 - iy1, 0.0)
        inter = iw * ih
        ua = area_a + area_g - inter
        iou = inter / jnp.maximum(ua, 1e-8)
        iou = jnp.where(valid, iou, 0.0)

        lane_nt = jax.lax.broadcasted_iota(jnp.int32, (N, TA), 1)
        colmax = jnp.max(iou, axis=1, keepdims=True)
        colarg = jnp.min(jnp.where(iou == colmax, lane_nt, _BIG_I32),
                         axis=1, keepdims=True) + tile * TA
        old = gtmax_ref[...]
        upd = colmax > old
        gtmax_ref[...] = jnp.where(upd, colmax, old)
        gtidx_ref[...] = jnp.where(upd, colarg, gtidx_ref[...])

        gt_rows = jax.lax.broadcasted_iota(jnp.int32, (N, TA), 0)
        iou_max = jnp.max(iou, axis=0, keepdims=True)
        iou_arg = jnp.min(jnp.where(iou == iou_max, gt_rows, _BIG_I32),
                          axis=0, keepdims=True)
        cache_ref[tile] = jnp.concatenate(
            [acx, acy, aw, ah, aang, iou_max,
             iou_arg.astype(jnp.float32), jnp.zeros((1, TA), jnp.float32)],
            axis=0)

    @pl.when(s >= T)
    def _pass2():
        slab = cache_ref[tile]
        acx, acy = slab[0:1, :], slab[1:2, :]
        aw, ah = slab[2:3, :], slab[3:4, :]
        aang = slab[4:5, :]
        iou_max = slab[5:6, :]
        iou_arg = slab[6:7, :].astype(jnp.int32)

        forced = jnp.any((gtidx_ref[...] == gid) & (gtmax_ref[...] < _MD_THRES),
                         axis=0, keepdims=True)
        positive = ((iou_max >= _MD_THRES) | forced) & valid

        gt_row = jax.lax.broadcasted_iota(jnp.int32, (N, 1), 0)
        onehot = (gt_row == iou_arg).astype(jnp.float32)
        tblT = jnp.transpose(jnp.concatenate(
            [gcx, gcy, gw, gh, gang, glbl], axis=1))
        assigned = jax.lax.dot_general(
            tblT, onehot, (((1,), (0,)), ((), ())),
            preferred_element_type=jnp.float32)
        a_cx, a_cy = assigned[0:1, :], assigned[1:2, :]
        a_w, a_h = assigned[2:3, :], assigned[3:4, :]
        a_ang = assigned[4:5, :]
        lbl = (assigned[5:6, :] + 0.5).astype(jnp.int32)

        c = jnp.clip(jnp.transpose(cls_ref[0]), 1e-4, 1.0 - 1e-4)
        f0 = ((1.0 - _ALPHA) * c * c) * (-jnp.log(1.0 - c + 1e-6))
        S0 = jnp.sum(f0, axis=0, keepdims=True)
        cid = jax.lax.broadcasted_iota(jnp.int32, (C, 1), 0)
        c_lbl = jnp.sum(jnp.where(lbl == cid, c, 0.0), axis=0, keepdims=True)
        f0_lbl = ((1.0 - _ALPHA) * c_lbl * c_lbl) * (-jnp.log(1.0 - c_lbl + 1e-6))
        om = 1.0 - c_lbl
        f1_lbl = (_ALPHA * om * om) * (-jnp.log(c_lbl + 1e-6))
        include = ((iou_max < _NEG_THRES) | positive) & valid
        cls_part = (jnp.sum(jnp.where(include, S0, 0.0))
                    + jnp.sum(jnp.where(positive, f1_lbl - f0_lbl, 0.0)))

        reg = jnp.transpose(reg_ref[0])
        dx = (a_cx - acx) / aw
        dy = (a_cy - acy) / ah
        dwc = jnp.log(a_w / aw)
        dhc = jnp.log(a_h / ah)
        dt = (a_ang - aang) * 3.141592653589793 / 180.0
        rt = jnp.concatenate([dx, dy, dwc, dhc, dt], axis=0)
        d = jnp.abs(reg - rt)
        rl = jnp.where(d < _BETA, 0.5 * d * d / _BETA, d - 0.5 * _BETA)
        rl = jnp.where(positive, rl, 0.0)

        acc_ref[0] = acc_ref[0] + cls_part
        acc_ref[1] = acc_ref[1] + jnp.sum(rl)
        acc_ref[2] = acc_ref[2] + jnp.sum(positive.astype(jnp.float32))

    @pl.when(s == 2 * T - 1)
    def _finish_image():
        npos = acc_ref[2]
        den = jnp.maximum(npos, 1.0)
        acc_ref[3] = acc_ref[3] + acc_ref[0] / den
        acc_ref[4] = acc_ref[4] + jnp.where(npos > 0.0,
                                            acc_ref[1] / (den * 5.0), 0.0)

    @pl.when((s == 2 * T - 1) & (b == B - 1))
    def _write_out():
        out_cls_ref[...] = jnp.full((1, 1), acc_ref[3] / B, jnp.float32)
        out_reg_ref[...] = jnp.full((1, 1), acc_ref[4] / B, jnp.float32)


def kernel(classifications, regressions, anchors, refined_achors, annotations):
    del refined_achors
    B, A, C = classifications.shape
    N = annotations.shape[1]
    TA = 6784 if A >= 6784 else A
    T = (A + TA - 1) // TA

    body = functools.partial(_body, T, TA, A, B, N, C)
    out_cls, out_reg = pl.pallas_call(
        body,
        grid=(B, 2 * T),
        in_specs=[
            pl.BlockSpec((1, TA, C), lambda b, s: (b, jnp.maximum(s - T, 0), 0)),
            pl.BlockSpec((1, TA, 5), lambda b, s: (b, jnp.maximum(s - T, 0), 0)),
            pl.BlockSpec((1, TA, 5), lambda b, s: (b, jnp.minimum(s, T - 1), 0)),
            pl.BlockSpec((1, N, 6), lambda b, s: (b, 0, 0)),
        ],
        out_specs=[
            pl.BlockSpec((1, 1), lambda b, s: (0, 0)),
            pl.BlockSpec((1, 1), lambda b, s: (0, 0)),
        ],
        out_shape=[
            jax.ShapeDtypeStruct((1, 1), jnp.float32),
            jax.ShapeDtypeStruct((1, 1), jnp.float32),
        ],
        scratch_shapes=[
            pltpu.VMEM((T, 8, TA), jnp.float32),
            pltpu.VMEM((N, 1), jnp.float32),
            pltpu.VMEM((N, 1), jnp.int32),
            pltpu.SMEM((8,), jnp.float32),
        ],
    )(classifications, regressions, anchors, annotations)
    return (out_cls.reshape(1), out_reg.reshape(1))


# speedup vs baseline: 1.2187x; 1.0441x over previous
import functools

import jax
import jax.numpy as jnp
from jax.experimental import pallas as pl
from jax.experimental.pallas import tpu as pltpu

_ALPHA = 0.25
_BETA = 1.0 / 9
_MD_THRES = 0.5
_NEG_THRES = _MD_THRES - 0.1
_BIG_I32 = 2**30


def _body(T, TA, A, B, N, C,
          cls_ref, reg_ref, anc_ref, ann_ref,
          out_cls_ref, out_reg_ref,
          cache_ref, gtmax_ref, gtidx_ref, acc_ref):
    b = pl.program_id(0)
    s = pl.program_id(1)
    tile = jax.lax.rem(s, T)

    @pl.when(s == 0)
    def _init_image():
        gtmax_ref[...] = jnp.full((N, 1), -1.0, jnp.float32)
        gtidx_ref[...] = jnp.zeros((N, 1), jnp.int32)
        acc_ref[0] = 0.0
        acc_ref[1] = 0.0
        acc_ref[2] = 0.0

    @pl.when((s == 0) & (b == 0))
    def _init_batch():
        acc_ref[3] = 0.0
        acc_ref[4] = 0.0

    ann = ann_ref[0]
    gcx = (ann[:, 0:1] + ann[:, 2:3]) * 0.5
    gcy = (ann[:, 1:2] + ann[:, 3:4]) * 0.5
    gw = ann[:, 2:3] - ann[:, 0:1]
    gh = ann[:, 3:4] - ann[:, 1:2]
    gang = ann[:, 4:5]
    glbl = ann[:, 5:6]

    laneid = jax.lax.broadcasted_iota(jnp.int32, (1, TA), 1)
    gid = laneid + tile * TA
    valid = gid < A

    @pl.when(s < T)
    def _pass1():
        gx1 = gcx - gw * 0.5
        gx2 = gcx + gw * 0.5
        gy1 = gcy - gh * 0.5
        gy2 = gcy + gh * 0.5
        area_g = (gx2 - gx1) * (gy2 - gy1)

        anc = jnp.transpose(anc_ref[0])
        acx = (anc[0:1, :] + anc[2:3, :]) * 0.5
        acy = (anc[1:2, :] + anc[3:4, :]) * 0.5
        aw = anc[2:3, :] - anc[0:1, :]
        ah = anc[3:4, :] - anc[1:2, :]
        aang = anc[4:5, :]
        ax1 = acx - aw * 0.5
        ax2 = acx + aw * 0.5
        ay1 = acy - ah * 0.5
        ay2 = acy + ah * 0.5
        area_a = (ax2 - ax1) * (ay2 - ay1)

        ix1 = jnp.maximum(ax1, gx1)
        iy1 = jnp.maximum(ay1, gy1)
        ix2 = jnp.minimum(ax2, gx2)
        iy2 = jnp.minimum(ay2, gy2)
        iw = jnp.clip(ix2 - ix1, 0.0)
        ih = jnp.clip(iy2 - iy1, 0.0)
        inter = iw * ih
        ua = area_a + area_g - inter
        iou = inter / jnp.maximum(ua, 1e-8)
        iou = jnp.where(valid, iou, 0.0)

        lane_nt = jax.lax.broadcasted_iota(jnp.int32, (N, TA), 1)
        colmax = jnp.max(iou, axis=1, keepdims=True)
        colarg = jnp.min(jnp.where(iou == colmax, lane_nt, _BIG_I32),
                         axis=1, keepdims=True) + tile * TA
        old = gtmax_ref[...]
        upd = colmax > old
        gtmax_ref[...] = jnp.where(upd, colmax, old)
        gtidx_ref[...] = jnp.where(upd, colarg, gtidx_ref[...])

        gt_rows = jax.lax.broadcasted_iota(jnp.int32, (N, TA), 0)
        iou_max = jnp.max(iou, axis=0, keepdims=True)
        iou_arg = jnp.min(jnp.where(iou == iou_max, gt_rows, _BIG_I32),
                          axis=0, keepdims=True)
        cache_ref[tile] = jnp.concatenate(
            [acx, acy, aw, ah, aang, iou_max,
             iou_arg.astype(jnp.float32), jnp.zeros((1, TA), jnp.float32)],
            axis=0)

    @pl.when(s >= T)
    def _pass2():
        slab = cache_ref[tile]
        acx, acy = slab[0:1, :], slab[1:2, :]
        aw, ah = slab[2:3, :], slab[3:4, :]
        aang = slab[4:5, :]
        iou_max = slab[5:6, :]
        iou_arg = slab[6:7, :].astype(jnp.int32)

        forced = jnp.any((gtidx_ref[...] == gid) & (gtmax_ref[...] < _MD_THRES),
                         axis=0, keepdims=True)
        positive = ((iou_max >= _MD_THRES) | forced) & valid

        gt_row = jax.lax.broadcasted_iota(jnp.int32, (N, 1), 0)
        onehot = (gt_row == iou_arg).astype(jnp.float32)
        tblT = jnp.transpose(jnp.concatenate(
            [gcx, gcy, gw, gh, gang, glbl], axis=1))
        assigned = jax.lax.dot_general(
            tblT, onehot, (((1,), (0,)), ((), ())),
            preferred_element_type=jnp.float32)
        a_cx, a_cy = assigned[0:1, :], assigned[1:2, :]
        a_w, a_h = assigned[2:3, :], assigned[3:4, :]
        a_ang = assigned[4:5, :]
        lbl = (assigned[5:6, :] + 0.5).astype(jnp.int32)

        c = jnp.clip(jnp.transpose(cls_ref[0]), 1e-4, 1.0 - 1e-4)
        f0 = ((1.0 - _ALPHA) * c * c) * (-jnp.log(1.0 - c + 1e-6))
        S0 = jnp.sum(f0, axis=0, keepdims=True)
        cid = jax.lax.broadcasted_iota(jnp.int32, (C, 1), 0)
        c_lbl = jnp.sum(jnp.where(lbl == cid, c, 0.0), axis=0, keepdims=True)
        f0_lbl = ((1.0 - _ALPHA) * c_lbl * c_lbl) * (-jnp.log(1.0 - c_lbl + 1e-6))
        om = 1.0 - c_lbl
        f1_lbl = (_ALPHA * om * om) * (-jnp.log(c_lbl + 1e-6))
        include = ((iou_max < _NEG_THRES) | positive) & valid
        cls_part = (jnp.sum(jnp.where(include, S0, 0.0))
                    + jnp.sum(jnp.where(positive, f1_lbl - f0_lbl, 0.0)))

        reg = jnp.transpose(reg_ref[0])
        dx = (a_cx - acx) / aw
        dy = (a_cy - acy) / ah
        dwc = jnp.log(a_w / aw)
        dhc = jnp.log(a_h / ah)
        dt = (a_ang - aang) * 3.141592653589793 / 180.0
        rt = jnp.concatenate([dx, dy, dwc, dhc, dt], axis=0)
        d = jnp.abs(reg - rt)
        rl = jnp.where(d < _BETA, 0.5 * d * d / _BETA, d - 0.5 * _BETA)
        rl = jnp.where(positive, rl, 0.0)

        acc_ref[0] = acc_ref[0] + cls_part
        acc_ref[1] = acc_ref[1] + jnp.sum(rl)
        acc_ref[2] = acc_ref[2] + jnp.sum(positive.astype(jnp.float32))

    @pl.when(s == 2 * T - 1)
    def _finish_image():
        npos = acc_ref[2]
        den = jnp.maximum(npos, 1.0)
        acc_ref[3] = acc_ref[3] + acc_ref[0] / den
        acc_ref[4] = acc_ref[4] + jnp.where(npos > 0.0,
                                            acc_ref[1] / (den * 5.0), 0.0)

    @pl.when((s == 2 * T - 1) & (b == B - 1))
    def _write_out():
        out_cls_ref[...] = jnp.full((1, 1), acc_ref[3] / B, jnp.float32)
        out_reg_ref[...] = jnp.full((1, 1), acc_ref[4] / B, jnp.float32)


def kernel(classifications, regressions, anchors, refined_achors, annotations):
    del refined_achors
    B, A, C = classifications.shape
    N = annotations.shape[1]
    TA = 10240 if A >= 10240 else A
    T = (A + TA - 1) // TA

    body = functools.partial(_body, T, TA, A, B, N, C)
    out_cls, out_reg = pl.pallas_call(
        body,
        grid=(B, 2 * T),
        in_specs=[
            pl.BlockSpec((1, TA, C), lambda b, s: (b, jnp.maximum(s - T, 0), 0)),
            pl.BlockSpec((1, TA, 5), lambda b, s: (b, jnp.maximum(s - T, 0), 0)),
            pl.BlockSpec((1, TA, 5), lambda b, s: (b, jnp.minimum(s, T - 1), 0)),
            pl.BlockSpec((1, N, 6), lambda b, s: (b, 0, 0)),
        ],
        out_specs=[
            pl.BlockSpec((1, 1), lambda b, s: (0, 0)),
            pl.BlockSpec((1, 1), lambda b, s: (0, 0)),
        ],
        out_shape=[
            jax.ShapeDtypeStruct((1, 1), jnp.float32),
            jax.ShapeDtypeStruct((1, 1), jnp.float32),
        ],
        scratch_shapes=[
            pltpu.VMEM((T, 8, TA), jnp.float32),
            pltpu.VMEM((N, 1), jnp.float32),
            pltpu.VMEM((N, 1), jnp.int32),
            pltpu.SMEM((8,), jnp.float32),
        ],
    )(classifications, regressions, anchors, annotations)
    return (out_cls.reshape(1), out_reg.reshape(1))


# speedup vs baseline: 1.2255x; 1.0055x over previous
import functools

import jax
import jax.numpy as jnp
from jax.experimental import pallas as pl
from jax.experimental.pallas import tpu as pltpu

_ALPHA = 0.25
_BETA = 1.0 / 9
_MD_THRES = 0.5
_NEG_THRES = _MD_THRES - 0.1
_BIG_I32 = 2**30


def _body(T, TA, A, B, N, C,
          cls_ref, reg_ref, anc_ref, ann_ref,
          out_cls_ref, out_reg_ref,
          cache_ref, gtmax_ref, gtidx_ref, acc_ref):
    b = pl.program_id(0)
    s = pl.program_id(1)
    tile = jax.lax.rem(s, T)

    @pl.when(s == 0)
    def _init_image():
        gtmax_ref[...] = jnp.full((N, 1), -1.0, jnp.float32)
        gtidx_ref[...] = jnp.zeros((N, 1), jnp.int32)
        acc_ref[0] = 0.0
        acc_ref[1] = 0.0
        acc_ref[2] = 0.0

    @pl.when((s == 0) & (b == 0))
    def _init_batch():
        acc_ref[3] = 0.0
        acc_ref[4] = 0.0

    ann = ann_ref[0]
    gcx = (ann[:, 0:1] + ann[:, 2:3]) * 0.5
    gcy = (ann[:, 1:2] + ann[:, 3:4]) * 0.5
    gw = ann[:, 2:3] - ann[:, 0:1]
    gh = ann[:, 3:4] - ann[:, 1:2]
    gang = ann[:, 4:5]
    glbl = ann[:, 5:6]

    laneid = jax.lax.broadcasted_iota(jnp.int32, (1, TA), 1)
    gid = laneid + tile * TA
    valid = gid < A

    @pl.when(s < T)
    def _pass1():
        gx1 = gcx - gw * 0.5
        gx2 = gcx + gw * 0.5
        gy1 = gcy - gh * 0.5
        gy2 = gcy + gh * 0.5
        area_g = (gx2 - gx1) * (gy2 - gy1)

        anc = jnp.transpose(anc_ref[0])
        acx = (anc[0:1, :] + anc[2:3, :]) * 0.5
        acy = (anc[1:2, :] + anc[3:4, :]) * 0.5
        aw = anc[2:3, :] - anc[0:1, :]
        ah = anc[3:4, :] - anc[1:2, :]
        aang = anc[4:5, :]
        ax1 = acx - aw * 0.5
        ax2 = acx + aw * 0.5
        ay1 = acy - ah * 0.5
        ay2 = acy + ah * 0.5
        area_a = (ax2 - ax1) * (ay2 - ay1)

        ix1 = jnp.maximum(ax1, gx1)
        iy1 = jnp.maximum(ay1, gy1)
        ix2 = jnp.minimum(ax2, gx2)
        iy2 = jnp.minimum(ay2, gy2)
        iw = jnp.clip(ix2 - ix1, 0.0)
        ih = jnp.clip(iy2 - iy1, 0.0)
        inter = iw * ih
        ua = area_a + area_g - inter
        iou = inter / jnp.maximum(ua, 1e-8)
        iou = jnp.where(valid, iou, 0.0)

        lane_nt = jax.lax.broadcasted_iota(jnp.int32, (N, TA), 1)
        colmax = jnp.max(iou, axis=1, keepdims=True)
        colarg = jnp.min(jnp.where(iou == colmax, lane_nt, _BIG_I32),
                         axis=1, keepdims=True) + tile * TA
        old = gtmax_ref[...]
        upd = colmax > old
        gtmax_ref[...] = jnp.where(upd, colmax, old)
        gtidx_ref[...] = jnp.where(upd, colarg, gtidx_ref[...])

        gt_rows = jax.lax.broadcasted_iota(jnp.int32, (N, TA), 0)
        iou_max = jnp.max(iou, axis=0, keepdims=True)
        iou_arg = jnp.min(jnp.where(iou == iou_max, gt_rows, _BIG_I32),
                          axis=0, keepdims=True)
        cache_ref[tile] = jnp.concatenate(
            [acx, acy, aw, ah, aang, iou_max,
             iou_arg.astype(jnp.float32), jnp.zeros((1, TA), jnp.float32)],
            axis=0)

    @pl.when(s >= T)
    def _pass2():
        slab = cache_ref[tile]
        acx, acy = slab[0:1, :], slab[1:2, :]
        aw, ah = slab[2:3, :], slab[3:4, :]
        aang = slab[4:5, :]
        iou_max = slab[5:6, :]
        iou_arg = slab[6:7, :].astype(jnp.int32)

        forced = jnp.any((gtidx_ref[...] == gid) & (gtmax_ref[...] < _MD_THRES),
                         axis=0, keepdims=True)
        positive = ((iou_max >= _MD_THRES) | forced) & valid

        gt_row = jax.lax.broadcasted_iota(jnp.int32, (N, 1), 0)
        onehot = (gt_row == iou_arg).astype(jnp.float32)
        tblT = jnp.transpose(jnp.concatenate(
            [gcx, gcy, gw, gh, gang, glbl], axis=1))
        assigned = jax.lax.dot_general(
            tblT, onehot, (((1,), (0,)), ((), ())),
            preferred_element_type=jnp.float32)
        a_cx, a_cy = assigned[0:1, :], assigned[1:2, :]
        a_w, a_h = assigned[2:3, :], assigned[3:4, :]
        a_ang = assigned[4:5, :]
        lbl = (assigned[5:6, :] + 0.5).astype(jnp.int32)

        c = jnp.clip(jnp.transpose(cls_ref[0]), 1e-4, 1.0 - 1e-4)
        f0 = ((1.0 - _ALPHA) * c * c) * (-jnp.log(1.0 - c + 1e-6))
        S0 = jnp.sum(f0, axis=0, keepdims=True)
        cid = jax.lax.broadcasted_iota(jnp.int32, (C, 1), 0)
        c_lbl = jnp.sum(jnp.where(lbl == cid, c, 0.0), axis=0, keepdims=True)
        f0_lbl = ((1.0 - _ALPHA) * c_lbl * c_lbl) * (-jnp.log(1.0 - c_lbl + 1e-6))
        om = 1.0 - c_lbl
        f1_lbl = (_ALPHA * om * om) * (-jnp.log(c_lbl + 1e-6))
        include = ((iou_max < _NEG_THRES) | positive) & valid
        cls_part = (jnp.sum(jnp.where(include, S0, 0.0))
                    + jnp.sum(jnp.where(positive, f1_lbl - f0_lbl, 0.0)))

        reg = jnp.transpose(reg_ref[0])
        dx = (a_cx - acx) / aw
        dy = (a_cy - acy) / ah
        dwc = jnp.log(a_w / aw)
        dhc = jnp.log(a_h / ah)
        dt = (a_ang - aang) * 3.141592653589793 / 180.0
        rt = jnp.concatenate([dx, dy, dwc, dhc, dt], axis=0)
        d = jnp.abs(reg - rt)
        rl = jnp.where(d < _BETA, 0.5 * d * d / _BETA, d - 0.5 * _BETA)
        rl = jnp.where(positive, rl, 0.0)

        acc_ref[0] = acc_ref[0] + cls_part
        acc_ref[1] = acc_ref[1] + jnp.sum(rl)
        acc_ref[2] = acc_ref[2] + jnp.sum(positive.astype(jnp.float32))

    @pl.when(s == 2 * T - 1)
    def _finish_image():
        npos = acc_ref[2]
        den = jnp.maximum(npos, 1.0)
        acc_ref[3] = acc_ref[3] + acc_ref[0] / den
        acc_ref[4] = acc_ref[4] + jnp.where(npos > 0.0,
                                            acc_ref[1] / (den * 5.0), 0.0)

    @pl.when((s == 2 * T - 1) & (b == B - 1))
    def _write_out():
        out_cls_ref[...] = jnp.full((1, 1), acc_ref[3] / B, jnp.float32)
        out_reg_ref[...] = jnp.full((1, 1), acc_ref[4] / B, jnp.float32)


def kernel(classifications, regressions, anchors, refined_achors, annotations):
    del refined_achors
    B, A, C = classifications.shape
    N = annotations.shape[1]
    TA = 10112 if A >= 10112 else A
    T = (A + TA - 1) // TA

    body = functools.partial(_body, T, TA, A, B, N, C)
    out_cls, out_reg = pl.pallas_call(
        body,
        grid=(B, 2 * T),
        in_specs=[
            pl.BlockSpec((1, TA, C), lambda b, s: (b, jnp.maximum(s - T, 0), 0)),
            pl.BlockSpec((1, TA, 5), lambda b, s: (b, jnp.maximum(s - T, 0), 0)),
            pl.BlockSpec((1, TA, 5), lambda b, s: (b, jnp.minimum(s, T - 1), 0)),
            pl.BlockSpec((1, N, 6), lambda b, s: (b, 0, 0)),
        ],
        out_specs=[
            pl.BlockSpec((1, 1), lambda b, s: (0, 0)),
            pl.BlockSpec((1, 1), lambda b, s: (0, 0)),
        ],
        out_shape=[
            jax.ShapeDtypeStruct((1, 1), jnp.float32),
            jax.ShapeDtypeStruct((1, 1), jnp.float32),
        ],
        scratch_shapes=[
            pltpu.VMEM((T, 8, TA), jnp.float32),
            pltpu.VMEM((N, 1), jnp.float32),
            pltpu.VMEM((N, 1), jnp.int32),
            pltpu.SMEM((8,), jnp.float32),
        ],
    )(classifications, regressions, anchors, annotations)
    return (out_cls.reshape(1), out_reg.reshape(1))
